# baseline (speedup 1.0000x reference)
import functools

import jax
import jax.numpy as jnp
import numpy as np
from jax import lax
from jax.experimental import pallas as pl
from jax.experimental.pallas import tpu as pltpu
from jax.experimental.pallas import tpu_sc as plsc

_DIM = 32
_SCALE = float(np.sqrt(_DIM))

_NC, _NS = 2, 16
_NW = _NC * _NS

_L = 128
_S = 8
_A = _DIM // _S
_G = 4
_U_ROWS = _G * _L
_RUN = _G * _S * _L
_STAGE = _A * _RUN
_NB = 4


def _sc_gather_scale(xq, table, n_rows, n_cols):
    n_i0t = n_rows // _L
    gblocks = n_i0t // _G
    n_units = n_cols * gblocks
    units_per_w = n_units // _NW
    assert n_units % _NW == 0 and units_per_w % _NB == 0

    mesh = plsc.VectorSubcoreMesh(
        core_axis_name="c", subcore_axis_name="s",
        num_cores=_NC, num_subcores=_NS,
    )

    @functools.partial(
        pl.kernel,
        out_type=jax.ShapeDtypeStruct((n_cols, _A, n_i0t * _S * _L),
                                      jnp.float32),
        mesh=mesh,
        scratch_types=(
            [pltpu.VMEM((_G, _S, _L), jnp.int32) for _ in range(_NB)]
            + [pltpu.VMEM((_U_ROWS, _DIM), jnp.float32) for _ in range(_NB)]
            + [pltpu.VMEM((_STAGE,), jnp.float32) for _ in range(2)]
            + [pltpu.SemaphoreType.DMA for _ in range(2 * _NB + 2)]
        ),
        compiler_params=pltpu.CompilerParams(
            use_tc_tiling_on_sc=False, needs_layout_passes=False),
    )
    def k(xq_hbm, table_hbm, outq_hbm, *scratch):
        idx_bufs = scratch[:_NB]
        row_bufs = scratch[_NB:2 * _NB]
        stage_bufs = scratch[2 * _NB:2 * _NB + 2]
        isems = scratch[2 * _NB + 2:3 * _NB + 2]
        gsems = scratch[3 * _NB + 2:4 * _NB + 2]
        osems = scratch[4 * _NB + 2:]

        wid = lax.axis_index("s") * _NC + lax.axis_index("c")
        u_base = wid * units_per_w
        iota = lax.iota(jnp.int32, 16)
        half_pat = [
            jnp.full((16,), h * 16, jnp.int32) + iota for h in (0, 1)]
        half_const = [
            (hp // _S) * _RUN + (hp % _S) * _L for hp in half_pat]

        def unit_pos(u):
            i1 = u // gblocks
            g0 = (u % gblocks) * _G
            return i1, g0

        def fire_idx(u, buf):
            i1, g0 = unit_pos(u_base + u)
            pltpu.async_copy(
                xq_hbm.at[i1 // _S, pl.ds(g0, _G)], idx_bufs[buf],
                isems[buf])

        def wait_idx(buf):
            pltpu.make_async_copy(
                xq_hbm.at[0, pl.ds(0, _G)], idx_bufs[buf], isems[buf],
            ).wait()

        def fire_gathers(u, buf):
            i1, _ = unit_pos(u_base + u)
            s1 = i1 % _S
            for j in range(_G):
                pltpu.async_copy(
                    table_hbm.at[idx_bufs[buf].at[j, s1]],
                    row_bufs[buf].at[pl.ds(j * _L, _L)],
                    gsems[buf],
                )

        def drain_gathers(buf):
            pltpu.make_async_copy(
                table_hbm.at[pl.ds(0, _U_ROWS)], row_bufs[buf], gsems[buf],
            ).wait()

        def fire_out(u, buf):
            i1, g0 = unit_pos(u_base + u)
            woff = g0 * _S * _L
            for a in range(_A):
                pltpu.async_copy(
                    stage_bufs[buf].at[pl.ds(a * _RUN, _RUN)],
                    outq_hbm.at[i1, a, pl.ds(woff, _RUN)],
                    osems[buf])

        def drain_out(buf):
            pltpu.make_async_copy(
                stage_bufs[buf], outq_hbm.at[0, 0, pl.ds(0, _STAGE)],
                osems[buf],
            ).wait()

        def transpose(buf, sbuf):
            rows_v = row_bufs[buf]
            stage_v = stage_bufs[sbuf]

            @plsc.parallel_loop(0, _U_ROWS, 1, unroll=4)
            def tr_body(p):
                gg = p // _L
                l = p % _L
                base = jnp.full((16,), gg * (_S * _L) + l, jnp.int32)
                for h in (0, 1):
                    vals = rows_v[p, pl.ds(h * 16, 16)] * _SCALE
                    plsc.store_scatter(stage_v, [base + half_const[h]], vals)

        i1p, g0p = unit_pos(u_base)
        pltpu.sync_copy(xq_hbm.at[i1p // _S, pl.ds(g0p, _G)], idx_bufs[0])
        fire_gathers(0, 0)
        for w in range(1, _NB):
            fire_idx(w, w)
        wait_idx(1)
        fire_gathers(1, 1)
        wait_idx(2)
        fire_gathers(2, 2)

        n = units_per_w

        def quad_body(q, carry):
            for b in range(_NB):
                u = q * _NB + b
                sb = b % 2
                drain_gathers(b)
                nb3 = (b + 3) % _NB
                @pl.when(u + 3 < n)
                def _():
                    wait_idx(nb3)
                    fire_gathers(u + 3, nb3)
                @pl.when(u + _NB < n)
                def _():
                    fire_idx(u + _NB, b)
                @pl.when(u >= 2)
                def _():
                    drain_out(sb)
                transpose(b, sb)
                fire_out(u, sb)
            return carry

        lax.fori_loop(0, n // _NB, quad_body, 0)
        drain_out(0)
        drain_out(1)

    return k(xq, table)


def kernel(x, table):
    n_rows, n_cols = x.shape
    xq = x.reshape(n_rows // _L, _L, n_cols // _S, _S).transpose(2, 0, 3, 1)
    outq = _sc_gather_scale(xq, table, n_rows, n_cols)
    outq = outq.reshape(n_cols, _A, n_rows // _L, _S, _L)
    out = outq.transpose(2, 4, 0, 1, 3).reshape(n_rows, n_cols, _DIM)
    return out


# speedup vs baseline: 2.3851x; 2.3851x over previous
import functools

import jax
import jax.numpy as jnp
import numpy as np
from jax import lax
from jax.experimental import pallas as pl
from jax.experimental.pallas import tpu as pltpu
from jax.experimental.pallas import tpu_sc as plsc

_DIM = 32
_SCALE = float(np.sqrt(_DIM))

_NC, _NS = 2, 16
_NW = _NC * _NS

_L = 128
_S = 8
_A = _DIM // _S
_G = 4
_U_ROWS = _G * _L
_RUN = _G * _S * _L
_STAGE = _A * _RUN
_SSTRIDE = _U_ROWS + 1
_ASTRIDE = _S * _SSTRIDE + _S
_IW = _A * _ASTRIDE


def _sc_gather_scale(xq, table, n_rows, n_cols):
    n_i0t = n_rows // _L
    gblocks = n_i0t // _G
    n_units = n_cols * gblocks
    units_per_w = n_units // _NW
    assert n_units % _NW == 0 and units_per_w % 4 == 0

    mesh = plsc.VectorSubcoreMesh(
        core_axis_name="c", subcore_axis_name="s",
        num_cores=_NC, num_subcores=_NS,
    )

    @functools.partial(
        pl.kernel,
        out_type=jax.ShapeDtypeStruct((n_cols, _A, n_i0t * _S * _L),
                                      jnp.float32),
        mesh=mesh,
        scratch_types=(
            [pltpu.VMEM((_G, _L), jnp.int32) for _ in range(4)]
            + [pltpu.VMEM((_U_ROWS, _DIM), jnp.float32) for _ in range(4)]
            + [pltpu.VMEM((_STAGE,), jnp.float32) for _ in range(2)]
            + [pltpu.VMEM((_IW,), jnp.float32)]
            + [pltpu.SMEM((_STAGE // 16,), jnp.int32)]
            + [pltpu.SemaphoreType.DMA for _ in range(4 + 4 + 2)]
        ),
        compiler_params=pltpu.CompilerParams(
            use_tc_tiling_on_sc=False, needs_layout_passes=False),
    )
    def k(xq_hbm, table_hbm, outq_hbm, *scratch):
        idx_bufs = scratch[0:4]
        row_bufs = scratch[4:8]
        stage_bufs = scratch[8:10]
        skew_v = scratch[10]
        src_tab = scratch[11]
        isems = scratch[12:16]
        gsems = scratch[16:20]
        osems = scratch[20:22]

        wid = lax.axis_index("s") * _NC + lax.axis_index("c")
        u_base = wid * units_per_w
        iota = lax.iota(jnp.int32, 16)
        half_pat = [iota + 16 * h for h in (0, 1)]
        skew_const = [
            (hp // _S) * _ASTRIDE + (hp % _S) * _SSTRIDE for hp in half_pat]

        def pre_body(j, carry):
            t = j // 8
            lv = j % 8
            a = t // 32
            gg = (t // 8) % _G
            s = t % 8
            src_tab[j] = (a * _ASTRIDE + s * _SSTRIDE + gg * _L + lv * 16)
            return carry

        lax.fori_loop(0, _STAGE // 16, pre_body, 0)

        def unit_pos(u):
            i1 = u // gblocks
            g0 = (u % gblocks) * _G
            return i1, g0

        def fire_idx(u, buf):
            i1, g0 = unit_pos(u_base + u)
            rt, s1 = i1 // _S, i1 % _S
            for j in range(_G):
                pltpu.async_copy(
                    xq_hbm.at[rt, g0 + j, s1], idx_bufs[buf].at[j],
                    isems[buf])

        def wait_idx(buf):
            for j in range(_G):
                pltpu.make_async_copy(
                    xq_hbm.at[0, 0, 0], idx_bufs[buf].at[j], isems[buf],
                ).wait()

        def fire_gathers(u, buf):
            for j in range(_G):
                pltpu.async_copy(
                    table_hbm.at[idx_bufs[buf].at[j]],
                    row_bufs[buf].at[pl.ds(j * _L, _L)],
                    gsems[buf],
                )

        def drain_gathers(buf):
            pltpu.make_async_copy(
                table_hbm.at[pl.ds(0, _U_ROWS)], row_bufs[buf], gsems[buf],
            ).wait()

        def fire_out(u, buf):
            i1, g0 = unit_pos(u_base + u)
            woff = g0 * _S * _L
            for a in range(_A):
                pltpu.async_copy(
                    stage_bufs[buf].at[pl.ds(a * _RUN, _RUN)],
                    outq_hbm.at[i1, a, pl.ds(woff, _RUN)],
                    osems[buf])

        def drain_out(buf):
            pltpu.make_async_copy(
                stage_bufs[buf], outq_hbm.at[0, 0, pl.ds(0, _STAGE)],
                osems[buf],
            ).wait()

        def transpose(buf, sbuf):
            rows_v = row_bufs[buf]
            stage_v = stage_bufs[sbuf]

            @plsc.parallel_loop(0, _U_ROWS, 1, unroll=4)
            def pass1(p):
                base = jnp.full((16,), p, jnp.int32)
                for h in (0, 1):
                    vals = rows_v[p, pl.ds(h * 16, 16)]
                    plsc.store_scatter(skew_v, [base + skew_const[h]], vals)

            @plsc.parallel_loop(0, _STAGE // 16, 1, unroll=8)
            def pass2(j):
                src = src_tab[j]
                ivec = jnp.full((16,), src, jnp.int32) + iota
                vals = plsc.load_gather(skew_v, [ivec])
                stage_v[pl.ds(j * 16, 16)] = vals * _SCALE

        i1p, g0p = unit_pos(u_base)
        rtp, s1p = i1p // _S, i1p % _S
        for j in range(_G):
            pltpu.sync_copy(xq_hbm.at[rtp, g0p + j, s1p], idx_bufs[0].at[j])
        fire_gathers(0, 0)
        fire_idx(1, 1)
        fire_idx(2, 2)
        wait_idx(1)
        fire_gathers(1, 1)

        n = units_per_w

        def quad_body(q, carry):
            for b4 in range(4):
                u = q * 4 + b4
                sb = b4 % 2
                drain_gathers(b4)
                nb4 = (b4 + 2) % 4
                @pl.when(u + 2 < n)
                def _():
                    wait_idx(nb4)
                    fire_gathers(u + 2, nb4)
                @pl.when(u + 3 < n)
                def _():
                    fire_idx(u + 3, (b4 + 3) % 4)
                @pl.when(u >= 2)
                def _():
                    drain_out(sb)
                transpose(b4, sb)
                fire_out(u, sb)
            return carry

        lax.fori_loop(0, n // 4, quad_body, 0)
        drain_out(0)
        drain_out(1)

    return k(xq, table)


def kernel(x, table):
    n_rows, n_cols = x.shape
    xq = x.reshape(n_rows // _L, _L, n_cols // _S, _S).transpose(2, 0, 3, 1)
    outq = _sc_gather_scale(xq, table, n_rows, n_cols)
    outq = outq.reshape(n_cols, _A, n_rows // _L, _S, _L)
    out = outq.transpose(2, 4, 0, 1, 3).reshape(n_rows, n_cols, _DIM)
    return out


# speedup vs baseline: 2.4132x; 1.0118x over previous
import functools

import jax
import jax.numpy as jnp
import numpy as np
from jax import lax
from jax.experimental import pallas as pl
from jax.experimental.pallas import tpu as pltpu
from jax.experimental.pallas import tpu_sc as plsc

_DIM = 32
_SCALE = float(np.sqrt(_DIM))

_NC, _NS = 2, 16
_NW = _NC * _NS

_L = 128
_S = 8
_A = _DIM // _S
_G = 4
_U_ROWS = _G * _L
_RUN = _G * _S * _L
_STAGE = _A * _RUN
_SSTRIDE = _U_ROWS + 1
_ASTRIDE = _S * _SSTRIDE + _S
_IW = _A * _ASTRIDE


def _sc_gather_scale(xq, table, n_rows, n_cols):
    n_i0t = n_rows // _L
    gblocks = n_i0t // _G
    n_units = n_cols * gblocks
    units_per_w = n_units // _NW
    assert n_units % _NW == 0 and units_per_w % 4 == 0

    mesh = plsc.VectorSubcoreMesh(
        core_axis_name="c", subcore_axis_name="s",
        num_cores=_NC, num_subcores=_NS,
    )

    @functools.partial(
        pl.kernel,
        out_type=jax.ShapeDtypeStruct((n_cols, _A, n_i0t * _S * _L),
                                      jnp.float32),
        mesh=mesh,
        scratch_types=(
            [pltpu.VMEM((_G, _L), jnp.int32) for _ in range(4)]
            + [pltpu.VMEM((_U_ROWS, _DIM), jnp.float32) for _ in range(4)]
            + [pltpu.VMEM((_STAGE,), jnp.float32) for _ in range(2)]
            + [pltpu.VMEM((_IW,), jnp.float32)]
            + [pltpu.SMEM((_STAGE // 16,), jnp.int32)]
            + [pltpu.SemaphoreType.DMA for _ in range(4 + 4 + 2)]
        ),
        compiler_params=pltpu.CompilerParams(
            use_tc_tiling_on_sc=False, needs_layout_passes=False),
    )
    def k(xq_hbm, table_hbm, outq_hbm, *scratch):
        idx_bufs = scratch[0:4]
        row_bufs = scratch[4:8]
        stage_bufs = scratch[8:10]
        skew_v = scratch[10]
        src_tab = scratch[11]
        isems = scratch[12:16]
        gsems = scratch[16:20]
        osems = scratch[20:22]

        wid = lax.axis_index("s") * _NC + lax.axis_index("c")
        u_base = wid * units_per_w
        iota = lax.iota(jnp.int32, 16)
        half_pat = [iota + 16 * h for h in (0, 1)]
        skew_const = [
            (hp // _S) * _ASTRIDE + (hp % _S) * _SSTRIDE for hp in half_pat]

        def pre_body(j, carry):
            t = j // 8
            lv = j % 8
            a = t // 32
            gg = (t // 8) % _G
            s = t % 8
            src_tab[j] = (a * _ASTRIDE + s * _SSTRIDE + gg * _L + lv * 16)
            return carry

        lax.fori_loop(0, _STAGE // 16, pre_body, 0)

        def unit_pos(u):
            i1 = u // gblocks
            g0 = (u % gblocks) * _G
            return i1, g0

        def fire_idx(u, buf):
            i1, g0 = unit_pos(u_base + u)
            rt, s1 = i1 // _S, i1 % _S
            for j in range(_G):
                pltpu.async_copy(
                    xq_hbm.at[rt, g0 + j, s1], idx_bufs[buf].at[j],
                    isems[buf])

        def wait_idx(buf):
            for j in range(_G):
                pltpu.make_async_copy(
                    xq_hbm.at[0, 0, 0], idx_bufs[buf].at[j], isems[buf],
                ).wait()

        def fire_gathers(u, buf):
            for j in range(_G):
                pltpu.async_copy(
                    table_hbm.at[idx_bufs[buf].at[j]],
                    row_bufs[buf].at[pl.ds(j * _L, _L)],
                    gsems[buf],
                )

        def drain_gathers(buf):
            pltpu.make_async_copy(
                table_hbm.at[pl.ds(0, _U_ROWS)], row_bufs[buf], gsems[buf],
            ).wait()

        def fire_out(u, buf):
            i1, g0 = unit_pos(u_base + u)
            woff = g0 * _S * _L
            for a in range(_A):
                pltpu.async_copy(
                    stage_bufs[buf].at[pl.ds(a * _RUN, _RUN)],
                    outq_hbm.at[i1, a, pl.ds(woff, _RUN)],
                    osems[buf])

        def drain_out(buf):
            pltpu.make_async_copy(
                stage_bufs[buf], outq_hbm.at[0, 0, pl.ds(0, _STAGE)],
                osems[buf],
            ).wait()

        def transpose(buf, sbuf):
            rows_v = row_bufs[buf]
            stage_v = stage_bufs[sbuf]

            @plsc.parallel_loop(0, _U_ROWS, 1, unroll=4)
            def pass1(p):
                base = jnp.full((16,), p, jnp.int32)
                for h in (0, 1):
                    vals = rows_v[p, pl.ds(h * 16, 16)]
                    plsc.store_scatter(skew_v, [base + skew_const[h]], vals)

            @plsc.parallel_loop(0, _STAGE // 16, 1, unroll=8)
            def pass2(j):
                src = src_tab[j]
                ivec = jnp.full((16,), src, jnp.int32) + iota
                vals = plsc.load_gather(skew_v, [ivec])
                stage_v[pl.ds(j * 16, 16)] = vals * _SCALE

        i1p, g0p = unit_pos(u_base)
        rtp, s1p = i1p // _S, i1p % _S
        for j in range(_G):
            pltpu.sync_copy(xq_hbm.at[rtp, g0p + j, s1p], idx_bufs[0].at[j])
        fire_gathers(0, 0)
        fire_idx(1, 1)
        fire_idx(2, 2)
        wait_idx(1)
        fire_gathers(1, 1)

        n = units_per_w

        def quad_body(q, carry):
            for b4 in range(4):
                u = q * 4 + b4
                sb = b4 % 2
                drain_gathers(b4)
                nb4 = (b4 + 2) % 4
                @pl.when(u + 2 < n)
                def _():
                    wait_idx(nb4)
                    fire_gathers(u + 2, nb4)
                @pl.when(u + 3 < n)
                def _():
                    fire_idx(u + 3, (b4 + 3) % 4)
                @pl.when(u >= 2)
                def _():
                    drain_out(sb)
                transpose(b4, sb)
                fire_out(u, sb)
            return carry

        lax.fori_loop(0, n // 4, quad_body, 0)
        drain_out(0)
        drain_out(1)

    return k(xq, table)


def kernel(x, table):
    n_rows, n_cols = x.shape
    xq = x.reshape(n_rows // _L, _L, n_cols // _S, _S).transpose(2, 0, 3, 1)
    xq = xq * 4
    tablep = jnp.pad(table, ((0, 0), (0, 128 - _DIM))).reshape(-1, _DIM)
    outq = _sc_gather_scale(xq, tablep, n_rows, n_cols)
    outq = outq.reshape(n_cols, _A, n_rows // _L, _S, _L)
    out = outq.transpose(2, 4, 0, 1, 3).reshape(n_rows, n_cols, _DIM)
    return out
